# baseline (speedup 1.0000x reference)
import jax
import jax.numpy as jnp
from jax.experimental import pallas as pl

N = 10000
E = 320000
D = 128
NG = 128
NC = 10
K = 5000
BLK = 1024


def _mlp_body(x_ref, a_ref, w1_ref, b1_ref, w2_ref, b2_ref, o_ref):
    z = x_ref[...] + a_ref[...]
    z = jnp.maximum(jnp.dot(z, w1_ref[...], preferred_element_type=jnp.float32) + b1_ref[...], 0.0)
    o_ref[...] = jnp.maximum(jnp.dot(z, w2_ref[...], preferred_element_type=jnp.float32) + b2_ref[...], 0.0)


def _gin_mlp(x, agg, W1, b1, W2, b2):
    n = x.shape[0]
    npad = ((n + BLK - 1) // BLK) * BLK
    xp = jnp.pad(x, ((0, npad - n), (0, 0)))
    ap = jnp.pad(agg, ((0, npad - n), (0, 0)))
    out = pl.pallas_call(
        _mlp_body,
        grid=(npad // BLK,),
        in_specs=[
            pl.BlockSpec((BLK, D), lambda i: (i, 0)),
            pl.BlockSpec((BLK, D), lambda i: (i, 0)),
            pl.BlockSpec((D, D), lambda i: (0, 0)),
            pl.BlockSpec((1, D), lambda i: (0, 0)),
            pl.BlockSpec((D, D), lambda i: (0, 0)),
            pl.BlockSpec((1, D), lambda i: (0, 0)),
        ],
        out_specs=pl.BlockSpec((BLK, D), lambda i: (i, 0)),
        out_shape=jax.ShapeDtypeStruct((npad, D), jnp.float32),
    )(xp, ap, W1, b1.reshape(1, D), W2, b2.reshape(1, D))
    return out[:n]


def kernel(x, enc_W1, enc_b1, enc_W2, enc_b2, enc_W3, enc_b3, enc_W4, enc_b4,
           pool_lin_W, pool_lin_b, pool_att_W, pool_att_b,
           score_W1, score_b1, score_W2, score_W3,
           gnn_W1, gnn_b1, gnn_W2, gnn_b2, gnn_W3, gnn_b3, gnn_W4, gnn_b4,
           cls_W, cls_b, edge_index, batch):
    src, dst = edge_index[0], edge_index[1]

    agg = jax.ops.segment_sum(x[src], dst, num_segments=N)
    h = _gin_mlp(x, agg, enc_W1, enc_b1, enc_W2, enc_b2)
    agg = jax.ops.segment_sum(h[src], dst, num_segments=N)
    h = _gin_mlp(h, agg, enc_W3, enc_b3, enc_W4, enc_b4)

    loop = jnp.arange(N, dtype=src.dtype)
    s2 = jnp.concatenate([src, loop])
    d2 = jnp.concatenate([dst, loop])
    xpj = h[s2]
    xq = jax.ops.segment_max(xpj, d2, num_segments=N)
    xq = (xq @ pool_lin_W + pool_lin_b)[d2]
    sc = (jnp.concatenate([xq, xpj], axis=-1) @ pool_att_W + pool_att_b)[:, 0]
    sc = jnp.where(sc > 0, sc, 0.2 * sc)
    m = jax.ops.segment_max(sc, d2, num_segments=N)
    e = jnp.exp(sc - m[d2])
    den = jax.ops.segment_sum(e, d2, num_segments=N)
    attn = e / (den[d2] + 1e-16)
    xc = jax.ops.segment_sum(xpj * attn[:, None], d2, num_segments=N)

    t2 = (xc @ score_W2)[d2]
    t3 = (xc @ score_W3)[s2]
    fit = (xc @ score_W1 + score_b1)[:, 0] + jax.ops.segment_sum((t2 - t3)[:, 0], d2, num_segments=N)
    fitness = jax.nn.sigmoid(fit)
    topv, perm = jax.lax.top_k(fitness, K)
    px = xc[perm] * topv[:, None]
    pbatch = batch[perm]
    kept = jnp.zeros((N,), jnp.float32).at[perm].set(1.0)
    nid = jnp.zeros((N,), src.dtype).at[perm].set(jnp.arange(K, dtype=src.dtype))
    em = kept[src] * kept[dst]
    ew = jax.nn.sigmoid(jnp.where(em > 0.5, 1.0, -1e9))
    ps, pd = nid[src], nid[dst]

    agg = jax.ops.segment_sum(px[ps] * ew[:, None], pd, num_segments=K)
    g = _gin_mlp(px, agg, gnn_W1, gnn_b1, gnn_W2, gnn_b2)
    agg = jax.ops.segment_sum(g[ps] * ew[:, None], pd, num_segments=K)
    g = _gin_mlp(g, agg, gnn_W3, gnn_b3, gnn_W4, gnn_b4)

    sums = jax.ops.segment_sum(g, pbatch, num_segments=NG)
    cnt = jax.ops.segment_sum(jnp.ones((K,), jnp.float32), pbatch, num_segments=NG)
    readout = sums / jnp.maximum(cnt, 1.0)[:, None]
    return readout @ cls_W + cls_b
